# baseline (speedup 1.0000x reference)
import jax
import jax.numpy as jnp
from jax import lax
from jax.experimental import pallas as pl
from jax.experimental.pallas import tpu as pltpu
from jax.experimental.pallas import tpu_sc as plsc

_NC, _NS, _L = 2, 16, 16
_NW = _NC * _NS

_ROWS, _COLS = 16384, 200
_N = _ROWS * _COLS
_PER_W = _N // _NW
_VECS = _PER_W // _L
_MAGIC = 12582912.0


_C = _PER_W // 2


def _sc_body(x_hbm, y_hbm, o_hbm, y_v, xbuf, obuf, insem, outsem):
    wid = lax.axis_index("s") * _NC + lax.axis_index("c")
    base = wid * _PER_W
    pltpu.sync_copy(y_hbm, y_v)

    scale = jnp.float32(y_v.shape[0] - 1)

    _NSEG = 8
    _SEG = _C // _NSEG
    for c in range(_PER_W // _C):
        off = base + c * _C
        hs = []
        for s in range(_NSEG):
            hs.append(pltpu.async_copy(
                x_hbm.at[pl.ds(off + s * _SEG, _SEG)],
                xbuf.at[pl.ds(s * _SEG, _SEG)], insem))
        for h in hs:
            h.wait()
        hs = []
        for s in range(_NSEG):
            hs.append(pltpu.async_copy(
                xbuf.at[pl.ds(s * _SEG, _SEG)],
                o_hbm.at[pl.ds(off + s * _SEG, _SEG)], outsem))
        for h in hs:
            h.wait()


_sc_call = pl.kernel(
    _sc_body,
    out_type=jax.ShapeDtypeStruct((_N,), jnp.float32),
    mesh=plsc.VectorSubcoreMesh(core_axis_name="c", subcore_axis_name="s"),
    scratch_types=[
        pltpu.VMEM((128,), jnp.float32),
        pltpu.VMEM((_C,), jnp.float32),
        pltpu.VMEM((_C,), jnp.float32),
        pltpu.SemaphoreType.DMA,
        pltpu.SemaphoreType.DMA,
    ],
    compiler_params=pltpu.CompilerParams(needs_layout_passes=False),
)


def kernel(x, y):
    out = _sc_call(x.reshape(_N).astype(y.dtype), y)
    return out.reshape(x.shape)


# speedup vs baseline: 1.1071x; 1.1071x over previous
import jax
import jax.numpy as jnp
from jax import lax
from jax.experimental import pallas as pl
from jax.experimental.pallas import tpu as pltpu
from jax.experimental.pallas import tpu_sc as plsc

_NC, _NS, _L = 2, 16, 16
_NW = _NC * _NS

_ROWS, _COLS = 16384, 200
_N = _ROWS * _COLS
_PER_W = _N // _NW
_VECS = _PER_W // _L
_MAGIC = 12582912.0


_C = _PER_W // 2


def _sc_body(x_hbm, y_hbm, o_hbm, y_v, xbuf, obuf, insem, outsem):
    wid = lax.axis_index("s") * _NC + lax.axis_index("c")
    base = wid * _PER_W
    pltpu.sync_copy(y_hbm, y_v)

    scale = jnp.float32(y_v.shape[0] - 1)

    _NSEG = 1
    _SEG = 256
    for c in range(1):
        off = base + c * _C
        hs = []
        for s in range(_NSEG):
            hs.append(pltpu.async_copy(
                x_hbm.at[pl.ds(off + s * _SEG, _SEG)],
                xbuf.at[pl.ds(s * _SEG, _SEG)], insem))
        for h in hs:
            h.wait()
        hs = []
        for s in range(_NSEG):
            hs.append(pltpu.async_copy(
                xbuf.at[pl.ds(s * _SEG, _SEG)],
                o_hbm.at[pl.ds(off + s * _SEG, _SEG)], outsem))
        for h in hs:
            h.wait()


_sc_call = pl.kernel(
    _sc_body,
    out_type=jax.ShapeDtypeStruct((_N,), jnp.float32),
    mesh=plsc.VectorSubcoreMesh(core_axis_name="c", subcore_axis_name="s"),
    scratch_types=[
        pltpu.VMEM((128,), jnp.float32),
        pltpu.VMEM((_C,), jnp.float32),
        pltpu.VMEM((_C,), jnp.float32),
        pltpu.SemaphoreType.DMA,
        pltpu.SemaphoreType.DMA,
    ],
    compiler_params=pltpu.CompilerParams(needs_layout_passes=False),
)


def kernel(x, y):
    out = _sc_call(x.reshape(_N).astype(y.dtype), y)
    return out.reshape(x.shape)


# speedup vs baseline: 1.4943x; 1.3497x over previous
import jax
import jax.numpy as jnp
from jax import lax
from jax.experimental import pallas as pl
from jax.experimental.pallas import tpu as pltpu
from jax.experimental.pallas import tpu_sc as plsc

_NC, _NS, _L = 2, 16, 16
_NW = _NC * _NS

_ROWS, _COLS = 16384, 200
_RPW = _ROWS // _NW
_RC = 128
_NCH = _RPW // _RC
_MAGIC = 12582912.0
_OFFS = tuple(range(0, _COLS - _L + 1, _L)) + (_COLS - _L,)


def _sc_body(x_hbm, y_hbm, o_hbm, y_v, xbuf, obuf):
    wid = lax.axis_index("s") * _NC + lax.axis_index("c")
    rbase = wid * _RPW
    pltpu.sync_copy(y_hbm, y_v)

    scale = jnp.float32(y_v.shape[0] - 1)

    for c in range(_NCH):
        roff = rbase + c * _RC
        pltpu.sync_copy(x_hbm.at[pl.ds(roff, _RC)], xbuf)

        @plsc.parallel_loop(0, _RC, step=1, unroll=2)
        def body(r):
            for j in _OFFS:
                sl = (r, pl.ds(j, _L))
                t = (xbuf[sl] * scale + _MAGIC) - _MAGIC
                obuf[sl] = plsc.load_gather(y_v, [t.astype(jnp.int32)])

        pltpu.sync_copy(obuf, o_hbm.at[pl.ds(roff, _RC)])


_sc_call = pl.kernel(
    _sc_body,
    out_type=jax.ShapeDtypeStruct((_ROWS, _COLS), jnp.float32),
    mesh=plsc.VectorSubcoreMesh(core_axis_name="c", subcore_axis_name="s"),
    scratch_types=[
        pltpu.VMEM((128,), jnp.float32),
        pltpu.VMEM((_RC, _COLS), jnp.float32),
        pltpu.VMEM((_RC, _COLS), jnp.float32),
    ],
    compiler_params=pltpu.CompilerParams(needs_layout_passes=False),
)


def kernel(x, y):
    return _sc_call(x.astype(y.dtype), y)


# speedup vs baseline: 1.6820x; 1.1256x over previous
import jax
import jax.numpy as jnp
from jax import lax
from jax.experimental import pallas as pl
from jax.experimental.pallas import tpu as pltpu
from jax.experimental.pallas import tpu_sc as plsc

_NC, _NS, _L = 2, 16, 16
_NW = _NC * _NS

_ROWS, _COLS = 16384, 200
_RPW = _ROWS // _NW
_RC = 64
_NCH = _RPW // _RC
_MAGIC = 12582912.0
_OFFS = tuple(range(0, _COLS - _L + 1, _L)) + (_COLS - _L,)


def _sc_body(x_hbm, y_hbm, o_hbm, y_v, xb0, xb1, ob0, ob1, insem, outsem):
    wid = lax.axis_index("s") * _NC + lax.axis_index("c")
    rbase = wid * _RPW
    pltpu.sync_copy(y_hbm, y_v)

    scale = jnp.float32(y_v.shape[0] - 1)
    xbufs, obufs = (xb0, xb1), (ob0, ob1)
    in_h, out_h = {}, {}

    def start_in(c):
        in_h[c] = pltpu.async_copy(
            x_hbm.at[pl.ds(rbase + c * _RC, _RC)], xbufs[c % 2], insem)

    def start_out(c):
        out_h[c] = pltpu.async_copy(
            obufs[c % 2], o_hbm.at[pl.ds(rbase + c * _RC, _RC)], outsem)

    start_in(0)
    for c in range(_NCH):
        if c + 1 < _NCH:
            start_in(c + 1)
        in_h[c].wait()
        if c >= 2:
            out_h[c - 2].wait()
        xbuf, obuf = xbufs[c % 2], obufs[c % 2]

        @plsc.parallel_loop(0, _RC, step=1, unroll=2)
        def body(r):
            for j in _OFFS:
                sl = (r, pl.ds(j, _L))
                t = (xbuf[sl] * scale + _MAGIC) - _MAGIC
                obuf[sl] = plsc.load_gather(y_v, [t.astype(jnp.int32)])

        start_out(c)
    out_h[_NCH - 2].wait()
    out_h[_NCH - 1].wait()


_sc_call = pl.kernel(
    _sc_body,
    out_type=jax.ShapeDtypeStruct((_ROWS, _COLS), jnp.float32),
    mesh=plsc.VectorSubcoreMesh(core_axis_name="c", subcore_axis_name="s"),
    scratch_types=[
        pltpu.VMEM((128,), jnp.float32),
        pltpu.VMEM((_RC, _COLS), jnp.float32),
        pltpu.VMEM((_RC, _COLS), jnp.float32),
        pltpu.VMEM((_RC, _COLS), jnp.float32),
        pltpu.VMEM((_RC, _COLS), jnp.float32),
        pltpu.SemaphoreType.DMA,
        pltpu.SemaphoreType.DMA,
    ],
    compiler_params=pltpu.CompilerParams(needs_layout_passes=False),
)


def kernel(x, y):
    return _sc_call(x.astype(y.dtype), y)


# speedup vs baseline: 1.6870x; 1.0030x over previous
import jax
import jax.numpy as jnp
from jax import lax
from jax.experimental import pallas as pl
from jax.experimental.pallas import tpu as pltpu
from jax.experimental.pallas import tpu_sc as plsc

_NC, _NS, _L = 2, 16, 16
_NW = _NC * _NS

_ROWS, _COLS = 16384, 200
_RPW = _ROWS // _NW
_RC = 64
_NCH = _RPW // _RC
_MAGIC = 12582912.0
_OFFS = tuple(range(0, _COLS - _L + 1, _L)) + (_COLS - _L,)


def _sc_body(x_hbm, y_hbm, o_hbm, y_v, xb0, xb1, ob0, ob1, insem, outsem):
    wid = lax.axis_index("s") * _NC + lax.axis_index("c")
    rbase = wid * _RPW
    pltpu.sync_copy(y_hbm, y_v)

    scale = jnp.float32(y_v.shape[0] - 1)
    xbufs, obufs = (xb0, xb1), (ob0, ob1)
    in_h, out_h = {}, {}

    def start_in(c):
        in_h[c] = pltpu.async_copy(
            x_hbm.at[pl.ds(rbase + c * _RC, _RC)], xbufs[c % 2], insem)

    def start_out(c):
        out_h[c] = pltpu.async_copy(
            obufs[c % 2], o_hbm.at[pl.ds(rbase + c * _RC, _RC)], outsem)

    start_in(0)
    for c in range(_NCH):
        if c + 1 < _NCH:
            start_in(c + 1)
        in_h[c].wait()
        if c >= 2:
            out_h[c - 2].wait()
        xbuf, obuf = xbufs[c % 2], obufs[c % 2]

        @plsc.parallel_loop(0, _RC, step=1, unroll=2)
        def body(r):
            for j in _OFFS:
                sl = (r, pl.ds(j, _L))
                t = (xbuf[sl] * scale + _MAGIC) - _MAGIC
                obuf[sl] = plsc.load_gather(y_v, [t.astype(jnp.int32)])

        start_out(c)
    out_h[_NCH - 2].wait()
    out_h[_NCH - 1].wait()


_sc_call = pl.kernel(
    _sc_body,
    out_type=jax.ShapeDtypeStruct((_ROWS, _COLS), jnp.float32),
    mesh=plsc.VectorSubcoreMesh(core_axis_name="c", subcore_axis_name="s"),
    scratch_types=[
        pltpu.VMEM((128,), jnp.float32),
        pltpu.VMEM((_RC, _COLS), jnp.float32),
        pltpu.VMEM((_RC, _COLS), jnp.float32),
        pltpu.VMEM((_RC, _COLS), jnp.float32),
        pltpu.VMEM((_RC, _COLS), jnp.float32),
        pltpu.SemaphoreType.DMA,
        pltpu.SemaphoreType.DMA,
    ],
    compiler_params=pltpu.CompilerParams(
        needs_layout_passes=False,
        use_tc_tiling_on_sc=True,
    ),
)


def kernel(x, y):
    return _sc_call(x.astype(y.dtype), y)
